# baseline (speedup 1.0000x reference)
import functools

import jax
import jax.numpy as jnp
from jax import lax
from jax.experimental import pallas as pl
from jax.experimental.pallas import tpu as pltpu
from jax.experimental.pallas import tpu_sc as plsc

B = 16384
D = 512
V = 1000
NC = 2
NS = 16
L = 16
NW = NC * NS

SB = 5120
RW = SB // NW
R = 32
NCHUNK = RW // R

TB = B - SB
BR = 1024
NBLK = TB // BR

_mesh = plsc.VectorSubcoreMesh(core_axis_name="c", subcore_axis_name="s")


@functools.partial(
    pl.kernel,
    mesh=_mesh,
    out_type=jax.ShapeDtypeStruct((NW, L), jnp.float32),
    scratch_types=[
        pltpu.VMEM((RW,), jnp.int32),
        pltpu.VMEM((R, D), jnp.float32),
        pltpu.VMEM((R, D), jnp.float32),
        pltpu.VMEM((R, D), jnp.float32),
        pltpu.VMEM((R, D), jnp.float32),
        pltpu.VMEM((R, D), jnp.float32),
        pltpu.VMEM((R, D), jnp.float32),
        pltpu.VMEM((L,), jnp.float32),
        pltpu.SemaphoreType.DMA,
        pltpu.SemaphoreType.DMA,
        pltpu.SemaphoreType.DMA,
        pltpu.SemaphoreType.DMA,
        pltpu.SemaphoreType.DMA,
        pltpu.SemaphoreType.DMA,
    ],
)
def _center_loss_sc(feats_hbm, targets_hbm, centers_hbm, out_hbm,
                    idx_all, f0, f1, f2, c0, c1, c2, acc_v,
                    semf0, semf1, semf2, semg0, semg1, semg2):
    wid = lax.axis_index("s") * NC + lax.axis_index("c")
    base = wid * RW

    NBUF = 3
    f_bufs = (f0, f1, f2)
    c_bufs = (c0, c1, c2)
    semf = (semf0, semf1, semf2)
    semg = (semg0, semg1, semg2)

    pltpu.sync_copy(targets_hbm.at[pl.ds(base, RW)], idx_all)

    def start(ch):
        b = ch % NBUF
        row0 = base + ch * R
        fcp = pltpu.async_copy(feats_hbm.at[pl.ds(row0, R)], f_bufs[b], semf[b])
        gcp = pltpu.async_copy(centers_hbm.at[idx_all.at[pl.ds(ch * R, R)]],
                               c_bufs[b], semg[b])
        return fcp, gcp

    inflight = [start(0), start(1), start(2)]

    NACC = 4
    accs = tuple(jnp.zeros((L,), jnp.float32) for _ in range(NACC))
    for ch in range(NCHUNK):
        b = ch % NBUF
        fcp, gcp = inflight[b]
        fcp.wait()
        gcp.wait()
        f_v = f_bufs[b]
        c_v = c_bufs[b]

        def row_body(r, accs):
            accs = list(accs)
            for j in range(D // L):
                d = f_v[r, pl.ds(j * L, L)] - c_v[r, pl.ds(j * L, L)]
                a = j % NACC
                accs[a] = accs[a] + d * d
            return tuple(accs)

        accs = lax.fori_loop(0, R, row_body, accs)
        if ch + NBUF < NCHUNK:
            inflight[b] = start(ch + NBUF)

    acc = accs[0]
    for a in accs[1:]:
        acc = acc + a
    acc_v[...] = acc
    pltpu.sync_copy(acc_v, out_hbm.at[wid])


def _tc_body(feats_ref, targets_ref, centers_ref, out_ref):
    step = pl.program_id(0)

    @pl.when(step == 0)
    def _init():
        out_ref[...] = jnp.zeros_like(out_ref)

    t = targets_ref[0, 0, :]
    cls = lax.broadcasted_iota(jnp.int32, (BR, V), 1)
    onehot = (t[:, None] == cls).astype(jnp.bfloat16)
    c_bf = centers_ref[...].astype(jnp.bfloat16)
    g = lax.dot_general(
        onehot, c_bf, (((1,), (0,)), ((), ())),
        preferred_element_type=jnp.float32)
    d = feats_ref[...] - g
    out_ref[...] += jnp.sum(d * d).reshape(1, 1)


def _center_loss_tc(feats, targets_2d, centers):
    return pl.pallas_call(
        _tc_body,
        grid=(NBLK,),
        in_specs=[
            pl.BlockSpec((BR, D), lambda i: (SB // BR + i, 0)),
            pl.BlockSpec((1, 1, BR), lambda i: (SB // BR + i, 0, 0)),
            pl.BlockSpec((V, D), lambda i: (0, 0)),
        ],
        out_specs=pl.BlockSpec((1, 1), lambda i: (0, 0)),
        out_shape=jax.ShapeDtypeStruct((1, 1), jnp.float32),
    )(feats, targets_2d, centers)


def kernel(feats, targets, centers):
    targets = targets.astype(jnp.int32)
    sc_part = _center_loss_sc(feats, targets, centers)
    tc_part = _center_loss_tc(feats, targets.reshape(B // BR, 1, BR), centers)
    return 0.5 * (jnp.sum(sc_part) + tc_part[0, 0]) / B


# speedup vs baseline: 1.0301x; 1.0301x over previous
import functools

import jax
import jax.numpy as jnp
from jax import lax
from jax.experimental import pallas as pl
from jax.experimental.pallas import tpu as pltpu
from jax.experimental.pallas import tpu_sc as plsc

B = 16384
D = 512
V = 1000
NC = 2
NS = 16
L = 16
NW = NC * NS

SB = 6144
RW = SB // NW
R = 32
NCHUNK = RW // R

TB = B - SB
BR = 1024
NBLK = TB // BR

_mesh = plsc.VectorSubcoreMesh(core_axis_name="c", subcore_axis_name="s")


@functools.partial(
    pl.kernel,
    mesh=_mesh,
    out_type=jax.ShapeDtypeStruct((NW, L), jnp.float32),
    scratch_types=[
        pltpu.VMEM((RW,), jnp.int32),
        pltpu.VMEM((R, D), jnp.float32),
        pltpu.VMEM((R, D), jnp.float32),
        pltpu.VMEM((R, D), jnp.float32),
        pltpu.VMEM((R, D), jnp.float32),
        pltpu.VMEM((R, D), jnp.float32),
        pltpu.VMEM((R, D), jnp.float32),
        pltpu.VMEM((L,), jnp.float32),
        pltpu.SemaphoreType.DMA,
        pltpu.SemaphoreType.DMA,
        pltpu.SemaphoreType.DMA,
        pltpu.SemaphoreType.DMA,
        pltpu.SemaphoreType.DMA,
        pltpu.SemaphoreType.DMA,
    ],
)
def _center_loss_sc(feats_hbm, targets_hbm, centers_hbm, out_hbm,
                    idx_all, f0, f1, f2, c0, c1, c2, acc_v,
                    semf0, semf1, semf2, semg0, semg1, semg2):
    wid = lax.axis_index("s") * NC + lax.axis_index("c")
    base = wid * RW

    NBUF = 3
    f_bufs = (f0, f1, f2)
    c_bufs = (c0, c1, c2)
    semf = (semf0, semf1, semf2)
    semg = (semg0, semg1, semg2)

    pltpu.sync_copy(targets_hbm.at[pl.ds(base, RW)], idx_all)

    def start(ch):
        b = ch % NBUF
        row0 = base + ch * R
        fcp = pltpu.async_copy(feats_hbm.at[pl.ds(row0, R)], f_bufs[b], semf[b])
        gcp = pltpu.async_copy(centers_hbm.at[idx_all.at[pl.ds(ch * R, R)]],
                               c_bufs[b], semg[b])
        return fcp, gcp

    inflight = [start(0), start(1), start(2)]

    NACC = 4
    accs = tuple(jnp.zeros((L,), jnp.float32) for _ in range(NACC))
    for ch in range(NCHUNK):
        b = ch % NBUF
        fcp, gcp = inflight[b]
        fcp.wait()
        gcp.wait()
        f_v = f_bufs[b]
        c_v = c_bufs[b]

        def row_body(r, accs):
            accs = list(accs)
            for j in range(D // L):
                d = f_v[r, pl.ds(j * L, L)] - c_v[r, pl.ds(j * L, L)]
                a = j % NACC
                accs[a] = accs[a] + d * d
            return tuple(accs)

        accs = lax.fori_loop(0, R, row_body, accs)
        if ch + NBUF < NCHUNK:
            inflight[b] = start(ch + NBUF)

    acc = accs[0]
    for a in accs[1:]:
        acc = acc + a
    acc_v[...] = acc
    pltpu.sync_copy(acc_v, out_hbm.at[wid])


def _tc_body(feats_ref, targets_ref, centers_ref, out_ref):
    step = pl.program_id(0)

    @pl.when(step == 0)
    def _init():
        out_ref[...] = jnp.zeros_like(out_ref)

    t = targets_ref[0, 0, :]
    cls = lax.broadcasted_iota(jnp.int32, (BR, V), 1)
    onehot = (t[:, None] == cls).astype(jnp.bfloat16)
    c_bf = centers_ref[...].astype(jnp.bfloat16)
    g = lax.dot_general(
        onehot, c_bf, (((1,), (0,)), ((), ())),
        preferred_element_type=jnp.float32)
    d = feats_ref[...] - g
    out_ref[...] += jnp.sum(d * d).reshape(1, 1)


def _center_loss_tc(feats, targets_2d, centers):
    return pl.pallas_call(
        _tc_body,
        grid=(NBLK,),
        in_specs=[
            pl.BlockSpec((BR, D), lambda i: (SB // BR + i, 0)),
            pl.BlockSpec((1, 1, BR), lambda i: (SB // BR + i, 0, 0)),
            pl.BlockSpec((V, D), lambda i: (0, 0)),
        ],
        out_specs=pl.BlockSpec((1, 1), lambda i: (0, 0)),
        out_shape=jax.ShapeDtypeStruct((1, 1), jnp.float32),
    )(feats, targets_2d, centers)


def kernel(feats, targets, centers):
    targets = targets.astype(jnp.int32)
    sc_part = _center_loss_sc(feats, targets, centers)
    tc_part = _center_loss_tc(feats, targets.reshape(B // BR, 1, BR), centers)
    return 0.5 * (jnp.sum(sc_part) + tc_part[0, 0]) / B
